# baseline (speedup 1.0000x reference)
import jax
import jax.numpy as jnp
from jax.experimental import pallas as pl
from jax.experimental.pallas import tpu as pltpu
from jax.experimental.pallas import tpu_sc as plsc

_B, _H, _W = 16, 512, 512
_N = _B * _H * _W
_TOPK = int(_N * 0.1)
_NTHR = 16
_ROWS = _N // _W
_BLK = 512
_NBLK = _ROWS // _BLK

_NC, _NS = 2, 16
_SHIFT = 19
_NBKT = 4096
_HWORDS = _NBKT * 16
_CH = 8192
_PER_TILE = _N // _NS
_NCH = _PER_TILE // _CH


def _edge_stats_kernel(p_ref, t_ref, ep_ref, et_ref, stats_ref, acc_ref):
    i = pl.program_id(0)

    @pl.when(i == 0)
    def _():
        acc_ref[...] = jnp.zeros_like(acc_ref)

    def edges(a):
        z_row = jnp.zeros((1, _W), jnp.float32)
        up = jnp.concatenate([z_row, a[:-1, :]], axis=0)
        dn = jnp.concatenate([a[1:, :], z_row], axis=0)
        s = up + 2.0 * a + dn
        d = dn - up
        z_col = jnp.zeros((_H, 1), jnp.float32)
        ex = jnp.concatenate([s[:, 1:], z_col], axis=1) - \
            jnp.concatenate([z_col, s[:, :-1]], axis=1)
        ey = jnp.concatenate([z_col, d[:, :-1]], axis=1) + 2.0 * d + \
            jnp.concatenate([d[:, 1:], z_col], axis=1)
        return jnp.sqrt(ex * ex + ey * ey + 1e-6)

    ep = edges(p_ref[0])
    et = edges(t_ref[0])
    ep_ref[0] = ep
    et_ref[0] = et

    def lanesum(x):
        return jnp.sum(x.reshape(_H // 8, 8, _W), axis=0)

    acc_ref[0] += lanesum(ep)
    acc_ref[1] += lanesum(ep * ep)
    acc_ref[2] += lanesum(et)
    acc_ref[3] += lanesum(et * et)

    @pl.when(i == _B - 1)
    def _():
        for q in range(4):
            stats_ref[0, q] = jnp.sum(acc_ref[q])


def _thresholds(lohi_ref, i, include_hi):
    lo = lohi_ref[i, 0]
    hi = lohi_ref[i, 1]
    if include_hi:
        step = (hi - lo) // _NTHR
        us = [lo + step * j for j in range(1, _NTHR)] + [hi]
    else:
        step = (hi - lo) // (_NTHR + 1)
        us = [lo + step * j for j in range(1, _NTHR + 1)]
    return [jax.lax.bitcast_convert_type(u, jnp.float32) for u in us]


def _lanesum(x):
    return jnp.sum(x.reshape(_BLK // 8, 8, _W), axis=0)


def _count_kernel(lohi_ref, ep_ref, et_ref, cnt_ref, acc_ref):
    b = pl.program_id(0)

    @pl.when(b == 0)
    def _():
        acc_ref[...] = jnp.zeros_like(acc_ref)

    for i, ref in enumerate((ep_ref, et_ref)):
        x = ref[...]
        thrs = _thresholds(lohi_ref, i, include_hi=False)
        for j in range(_NTHR):
            mask = x > thrs[j]
            acc_ref[i, j] += _lanesum(mask.astype(jnp.float32))

    @pl.when(b == _NBLK - 1)
    def _():
        for i in range(2):
            for j in range(_NTHR):
                cnt_ref[i, j] = jnp.sum(acc_ref[i, j])


def _count_sum_kernel(lohi_ref, ep_ref, et_ref, cnt_ref, sm_ref, acc_ref,
                      sacc_ref):
    b = pl.program_id(0)

    @pl.when(b == 0)
    def _():
        acc_ref[...] = jnp.zeros_like(acc_ref)
        sacc_ref[...] = jnp.zeros_like(sacc_ref)

    for i, ref in enumerate((ep_ref, et_ref)):
        x = ref[...]
        thrs = _thresholds(lohi_ref, i, include_hi=True)
        for j in range(_NTHR):
            mask = x > thrs[j]
            acc_ref[i, j] += _lanesum(mask.astype(jnp.float32))
            sacc_ref[i, j] += _lanesum(jnp.where(mask, x, 0.0))

    @pl.when(b == _NBLK - 1)
    def _():
        for i in range(2):
            for j in range(_NTHR):
                cnt_ref[i, j] = jnp.sum(acc_ref[i, j])
                sm_ref[i, j] = jnp.sum(sacc_ref[i, j])


def _sc_hist_kernel(ep_ref, et_ref, out_ref, buf0, buf1, hist, sem0, sem1):
    c = jax.lax.axis_index("c")
    s = jax.lax.axis_index("s")
    wid = s * _NC + c

    def zbody(k, carry):
        hist[pl.ds(k * 16, 16)] = jnp.zeros((16,), jnp.float32)
        return carry

    jax.lax.fori_loop(0, _HWORDS // 16, zbody, 0, unroll=8)

    base = s * _PER_TILE
    ones = jnp.ones((16,), jnp.float32)
    lane = jax.lax.iota(jnp.int32, 16)

    def process(src_ref):
        bufs = (buf0, buf1)
        sems = (sem0, sem1)
        cps = [None, None]
        cps[0] = pltpu.async_copy(src_ref.at[pl.ds(base, _CH)], buf0, sem0)
        for ci in range(_NCH):
            pb = ci % 2
            cps[pb].wait()
            if ci + 1 < _NCH:
                nb = (ci + 1) % 2
                cps[nb] = pltpu.async_copy(
                    src_ref.at[pl.ds(base + (ci + 1) * _CH, _CH)],
                    bufs[nb], sems[nb])
            buf = bufs[pb]

            def body(k, carry):
                v = buf[pl.ds(k * 16, 16)]
                bits = plsc.bitcast(v, jnp.int32)
                bkt = jax.lax.shift_right_arithmetic(bits, _SHIFT)
                idx = jax.lax.shift_left(bkt, 4) + lane
                plsc.addupdate_scatter(hist, [idx], ones)
                return carry

            jax.lax.fori_loop(0, _CH // 16, body, 0, unroll=8)

    @pl.when(c == 0)
    def _():
        process(ep_ref)

    @pl.when(c == 1)
    def _():
        process(et_ref)

    pltpu.sync_copy(hist, out_ref.at[wid])


def _run_edges(p, t):
    return pl.pallas_call(
        _edge_stats_kernel,
        grid=(_B,),
        in_specs=[
            pl.BlockSpec((1, _H, _W), lambda i: (i, 0, 0)),
            pl.BlockSpec((1, _H, _W), lambda i: (i, 0, 0)),
        ],
        out_specs=[
            pl.BlockSpec((1, _H, _W), lambda i: (i, 0, 0)),
            pl.BlockSpec((1, _H, _W), lambda i: (i, 0, 0)),
            pl.BlockSpec(memory_space=pltpu.SMEM),
        ],
        out_shape=[
            jax.ShapeDtypeStruct((_B, _H, _W), jnp.float32),
            jax.ShapeDtypeStruct((_B, _H, _W), jnp.float32),
            jax.ShapeDtypeStruct((1, 4), jnp.float32),
        ],
        scratch_shapes=[pltpu.VMEM((4, 8, _W), jnp.float32)],
    )(p, t)


def _run_sc_hist(e1p, e1t):
    mesh = plsc.VectorSubcoreMesh(
        core_axis_name="c", subcore_axis_name="s",
        num_cores=_NC, num_subcores=_NS)
    return pl.kernel(
        _sc_hist_kernel,
        out_type=jax.ShapeDtypeStruct((_NC * _NS, _HWORDS), jnp.float32),
        mesh=mesh,
        compiler_params=pltpu.CompilerParams(needs_layout_passes=False),
        scratch_types=[
            pltpu.VMEM((_CH,), jnp.float32),
            pltpu.VMEM((_CH,), jnp.float32),
            pltpu.VMEM((_HWORDS,), jnp.float32),
            pltpu.SemaphoreType.DMA,
            pltpu.SemaphoreType.DMA,
        ],
    )(e1p, e1t)


def _run_count(lohi, e2p, e2t, with_sums):
    body = _count_sum_kernel if with_sums else _count_kernel
    n_out = 2 if with_sums else 1
    scratch = [pltpu.VMEM((2, _NTHR, 8, _W), jnp.float32)] * n_out
    return pl.pallas_call(
        body,
        grid=(_NBLK,),
        in_specs=[
            pl.BlockSpec(memory_space=pltpu.SMEM),
            pl.BlockSpec((_BLK, _W), lambda b: (b, 0)),
            pl.BlockSpec((_BLK, _W), lambda b: (b, 0)),
        ],
        out_specs=[pl.BlockSpec(memory_space=pltpu.SMEM)] * n_out,
        out_shape=[jax.ShapeDtypeStruct((2, _NTHR), jnp.float32)] * n_out,
        scratch_shapes=scratch,
    )(lohi, e2p, e2t)


def _interior_u(lo, hi, include_hi):
    j_idx = jnp.arange(1, _NTHR + 1, dtype=jnp.int32)
    if include_hi:
        step = (hi - lo) // _NTHR
        u = lo[:, None] + step[:, None] * j_idx[None, :]
        return u.at[:, _NTHR - 1].set(hi)
    step = (hi - lo) // (_NTHR + 1)
    return lo[:, None] + step[:, None] * j_idx[None, :]


def kernel(pred, target, source):
    p = pred.reshape(_B, _H, _W)
    t = target.reshape(_B, _H, _W)
    ep, et, stats = _run_edges(p, t)
    sums = stats[0]

    n_f = jnp.float32(_N)
    mean_p, mean_t = sums[0] / n_f, sums[2] / n_f
    var_p = (sums[1] - sums[0] * mean_p) / (n_f - 1.0)
    var_t = (sums[3] - sums[2] * mean_t) / (n_f - 1.0)
    stats_loss = jnp.abs(mean_p - mean_t) + jnp.abs(
        jnp.sqrt(var_p) - jnp.sqrt(var_t))

    e2p = ep.reshape(_ROWS, _W)
    e2t = et.reshape(_ROWS, _W)
    nk = jnp.float32(_TOPK)

    hist32 = _run_sc_hist(ep.reshape(_N), et.reshape(_N))
    h = jnp.sum(hist32.reshape(_NS, _NC, _NBKT, 16), axis=(0, 3))
    inc = jnp.cumsum(h[:, ::-1], axis=1)[:, ::-1]
    bkt = jnp.sum((inc >= nk).astype(jnp.int32), axis=1) - 1
    lo = (jnp.maximum(bkt - 1, 0) << _SHIFT) - 1
    hi = ((bkt + 2) << _SHIFT) - 1

    u = _interior_u(lo, hi, include_hi=False)
    (cnt,) = _run_count(jnp.stack([lo, hi], axis=1), e2p, e2t,
                        with_sums=False)
    ge = cnt >= nk
    lo = jnp.max(jnp.where(ge, u, lo[:, None]), axis=1)
    hi = jnp.min(jnp.where(ge, hi[:, None], u), axis=1)

    u = _interior_u(lo, hi, include_hi=True)
    cnt, sm = _run_count(jnp.stack([lo, hi], axis=1), e2p, e2t,
                         with_sums=True)
    ge = cnt >= nk
    first_lt = jnp.minimum(jnp.sum(ge.astype(jnp.int32), axis=1), _NTHR - 1)
    lo = jnp.max(jnp.where(ge, u, lo[:, None]), axis=1)
    hi = jnp.min(jnp.where(ge, hi[:, None], u), axis=1)
    cg_hi = jnp.take_along_axis(cnt, first_lt[:, None], axis=1)[:, 0]
    sg_hi = jnp.take_along_axis(sm, first_lt[:, None], axis=1)[:, 0]

    v_lo = jax.lax.bitcast_convert_type(lo, jnp.float32)
    v_hi = jax.lax.bitcast_convert_type(hi, jnp.float32)
    t_mid = 0.5 * (v_lo + v_hi)
    s_top = sg_hi + (nk - cg_hi) * t_mid
    topk_loss = jnp.abs(s_top[0] / nk - s_top[1] / nk)
    return (stats_loss + topk_loss).astype(jnp.float32)


# speedup vs baseline: 2.1861x; 2.1861x over previous
import jax
import jax.numpy as jnp
from jax.experimental import pallas as pl
from jax.experimental.pallas import tpu as pltpu

_B, _H, _W = 16, 512, 512
_N = _B * _H * _W
_TOPK = int(_N * 0.1)
_ROWS = _N // _W
_BLK = 512
_NBLK = _ROWS // _BLK

_NTHR_C = 4
_NPASS_C = 8
_NTHR_F = 8
_NPH = 1 + _NPASS_C + 1


def _edges(a):
    z_row = jnp.zeros((1, _W), jnp.float32)
    up = jnp.concatenate([z_row, a[:-1, :]], axis=0)
    dn = jnp.concatenate([a[1:, :], z_row], axis=0)
    s = up + 2.0 * a + dn
    d = dn - up
    z_col = jnp.zeros((_H, 1), jnp.float32)
    ex = jnp.concatenate([s[:, 1:], z_col], axis=1) - \
        jnp.concatenate([z_col, s[:, :-1]], axis=1)
    ey = jnp.concatenate([z_col, d[:, :-1]], axis=1) + 2.0 * d + \
        jnp.concatenate([d[:, 1:], z_col], axis=1)
    return jnp.sqrt(ex * ex + ey * ey + 1e-6)


def _lanesum(x):
    return jnp.sum(x.reshape(_BLK // 8, 8, _W), axis=0)


def _loss_kernel(p_ref, t_ref, out_ref,
                 e_p, e_t, acc, cacc, sacc, brk, res):
    ph = pl.program_id(0)
    b = pl.program_id(1)
    nk = jnp.float32(_TOPK)

    @pl.when((ph == 0) & (b == 0))
    def _():
        acc[...] = jnp.zeros_like(acc)
        for i in range(2):
            brk[i, 0] = 0
            brk[i, 1] = 0x7F7FFFFF

    @pl.when(ph == 0)
    def _():
        ep = _edges(p_ref[0])
        et = _edges(t_ref[0])
        e_p[pl.ds(b * _BLK, _BLK), :] = ep
        e_t[pl.ds(b * _BLK, _BLK), :] = et
        acc[0] += _lanesum(ep)
        acc[1] += _lanesum(ep * ep)
        acc[2] += _lanesum(et)
        acc[3] += _lanesum(et * et)

    def thresholds(i, nthr, include_hi):
        lo = brk[i, 0]
        hi = brk[i, 1]
        if include_hi:
            step = (hi - lo) // nthr
            us = [lo + step * j for j in range(1, nthr)] + [hi]
        else:
            step = (hi - lo) // (nthr + 1)
            us = [lo + step * j for j in range(1, nthr + 1)]
        return us, [jax.lax.bitcast_convert_type(u, jnp.float32) for u in us]

    def bracket_update(i, nthr, cnts, us):
        lo, hi = brk[i, 0], brk[i, 1]
        new_lo, new_hi = lo, hi
        for j in range(nthr - 1, -1, -1):
            ge = cnts[j] >= nk
            new_lo = jnp.where(ge, jnp.maximum(new_lo, us[j]), new_lo)
            new_hi = jnp.where(ge, new_hi, us[j])
        brk[i, 0] = new_lo
        brk[i, 1] = new_hi
        return new_lo, new_hi

    @pl.when((ph >= 1) & (ph <= _NPASS_C))
    def _():
        @pl.when(b == 0)
        def _():
            cacc[...] = jnp.zeros_like(cacc)

        for i, e in enumerate((e_p, e_t)):
            x = e[pl.ds(b * _BLK, _BLK), :]
            _, thrs = thresholds(i, _NTHR_C, include_hi=False)
            for j in range(_NTHR_C):
                cacc[i, j] += _lanesum((x > thrs[j]).astype(jnp.float32))

        @pl.when(b == _NBLK - 1)
        def _():
            for i in range(2):
                us, _ = thresholds(i, _NTHR_C, include_hi=False)
                cnts = [jnp.sum(cacc[i, j]) for j in range(_NTHR_C)]
                bracket_update(i, _NTHR_C, cnts, us)

    @pl.when(ph == _NPH - 1)
    def _():
        @pl.when(b == 0)
        def _():
            cacc[...] = jnp.zeros_like(cacc)
            sacc[...] = jnp.zeros_like(sacc)

        for i, e in enumerate((e_p, e_t)):
            x = e[pl.ds(b * _BLK, _BLK), :]
            _, thrs = thresholds(i, _NTHR_F, include_hi=True)
            for j in range(_NTHR_F):
                mask = x > thrs[j]
                cacc[i, j] += _lanesum(mask.astype(jnp.float32))
                sacc[i, j] += _lanesum(jnp.where(mask, x, 0.0))

        @pl.when(b == _NBLK - 1)
        def _():
            for i in range(2):
                us, _ = thresholds(i, _NTHR_F, include_hi=True)
                cnts = [jnp.sum(cacc[i, j]) for j in range(_NTHR_F)]
                sms = [jnp.sum(sacc[i, j]) for j in range(_NTHR_F)]
                new_lo, new_hi = bracket_update(i, _NTHR_F, cnts, us)
                cg = cnts[_NTHR_F - 1]
                sg = sms[_NTHR_F - 1]
                for j in range(_NTHR_F - 2, -1, -1):
                    lt = cnts[j] < nk
                    cg = jnp.where(lt, cnts[j], cg)
                    sg = jnp.where(lt, sms[j], sg)
                v_lo = jax.lax.bitcast_convert_type(new_lo, jnp.float32)
                v_hi = jax.lax.bitcast_convert_type(new_hi, jnp.float32)
                s_top = sg + (nk - cg) * 0.5 * (v_lo + v_hi)
                res[i] = s_top / nk

            n_f = jnp.float32(_N)
            sums = [jnp.sum(acc[q]) for q in range(4)]
            mean_p = sums[0] / n_f
            mean_t = sums[2] / n_f
            var_p = (sums[1] - sums[0] * mean_p) / (n_f - 1.0)
            var_t = (sums[3] - sums[2] * mean_t) / (n_f - 1.0)
            stats_loss = jnp.abs(mean_p - mean_t) + jnp.abs(
                jnp.sqrt(var_p) - jnp.sqrt(var_t))
            out_ref[0, 0] = stats_loss + jnp.abs(res[0] - res[1])


def kernel(pred, target, source):
    p = pred.reshape(_B, _H, _W)
    t = target.reshape(_B, _H, _W)

    def in_map(ph, b):
        return (jnp.where(ph == 0, b, _NBLK - 1), 0, 0)

    out = pl.pallas_call(
        _loss_kernel,
        grid=(_NPH, _NBLK),
        in_specs=[
            pl.BlockSpec((1, _H, _W), in_map),
            pl.BlockSpec((1, _H, _W), in_map),
        ],
        out_specs=pl.BlockSpec(memory_space=pltpu.SMEM),
        out_shape=jax.ShapeDtypeStruct((1, 1), jnp.float32),
        scratch_shapes=[
            pltpu.VMEM((_ROWS, _W), jnp.float32),
            pltpu.VMEM((_ROWS, _W), jnp.float32),
            pltpu.VMEM((4, 8, _W), jnp.float32),
            pltpu.VMEM((2, _NTHR_F, 8, _W), jnp.float32),
            pltpu.VMEM((2, _NTHR_F, 8, _W), jnp.float32),
            pltpu.SMEM((2, 2), jnp.int32),
            pltpu.SMEM((2,), jnp.float32),
        ],
    )(p, t)
    return out[0, 0]
